# baseline (speedup 1.0000x reference)
import functools

import jax
import jax.numpy as jnp
from jax import lax
from jax.experimental import pallas as pl
from jax.experimental.pallas import tpu as pltpu
from jax.experimental.pallas import tpu_sc as plsc

NU = 16
ND = 48
NN = 50000
NE = 1600000
NC, NS = 2, 16
NW = NC * NS
CHUNK = 256
HALF = 128
NCH = NE // CHUNK
CH_BASE = NCH // NW
CH_EXTRA = NCH - CH_BASE * NW
AWORDS = 802816
ZW = AWORDS // NS
_BCAST_DIMS = lax.GatherDimensionNumbers(
    offset_dims=(), collapsed_slice_dims=(0,), start_index_map=(0,))


def _sc_scatter_body(deltas, idx1, idx2, out, bout, d_v, uf1_v, if1_v,
                     uf2_v, if2_v, bf_v, i1_v, i2_v, sem_s, sem_b, acc):
    c = lax.axis_index("c")
    s = lax.axis_index("s")
    g = c * NS + s

    zero16 = jnp.zeros((16,), jnp.float32)

    @plsc.parallel_loop(0, HALF * NU // 16, unroll=8)
    def _(i):
        uf1_v[pl.ds(i * 16, 16)] = zero16

    z0 = s * ZW
    for q in range(24):
        pltpu.sync_copy(uf1_v, acc.at[pl.ds(z0 + q * 2048, 2048)])
    pltpu.sync_copy(uf1_v.at[pl.ds(0, ZW - 24 * 2048)],
                    acc.at[pl.ds(z0 + 24 * 2048, ZW - 24 * 2048)])
    plsc.subcore_barrier()

    nch = CH_BASE + jnp.where(g < CH_EXTRA, 1, 0)
    start = g * CH_BASE + jnp.minimum(g, CH_EXTRA)
    iota16 = lax.iota(jnp.int32, 16)

    def chunk_body(i, carry):
        base = (start + i) * CHUNK
        pltpu.sync_copy(idx1.at[pl.ds(base, CHUNK)], i1_v)
        pltpu.sync_copy(idx2.at[pl.ds(base, CHUNK)], i2_v)
        for h in range(2):
            e0 = base + h * HALF
            pltpu.sync_copy(deltas.at[pl.ds(e0, HALF)], d_v)

            @plsc.parallel_loop(0, HALF, unroll=8)
            def _(r):
                grp = (r >> 4) << 4
                lane = r - grp
                bvec = jnp.full((16, 1), lane, jnp.int32)
                i1v = i1_v[pl.ds(h * HALF + grp, 16)]
                n1 = lax.gather(i1v, bvec, _BCAST_DIMS, (1,),
                                mode=lax.GatherScatterMode.PROMISE_IN_BOUNDS)
                i2v = i2_v[pl.ds(h * HALF + grp, 16)]
                n2 = lax.gather(i2v, bvec, _BCAST_DIMS, (1,),
                                mode=lax.GatherScatterMode.PROMISE_IN_BOUNDS)
                uf1_v[pl.ds(r * 16, 16)] = d_v[r, 0:NU]
                if1_v[pl.ds(r * 16, 16)] = n1 * NU + iota16
                uf2_v[pl.ds(r * 16, 16)] = d_v[r, NU:2 * NU]
                if2_v[pl.ds(r * 16, 16)] = n2 * NU + iota16
                bf_v[r, :] = d_v[r, 2 * NU:3 * NU]

            d1 = pltpu.async_copy(uf1_v, acc.at[if1_v], sem_s, add=True)
            d2 = pltpu.async_copy(uf2_v, acc.at[if2_v], sem_s, add=True)
            d3 = pltpu.async_copy(bf_v, bout.at[pl.ds(e0, HALF)], sem_b)
            d1.wait()
            d2.wait()
            d3.wait()
        return carry

    lax.fori_loop(0, nch, chunk_body, 0)
    plsc.subcore_barrier()

    pltpu.sync_copy(acc.at[pl.ds(z0, ZW)], out.at[pl.ds(c * AWORDS + z0, ZW)])


_sc_scatter = functools.partial(
    pl.kernel,
    out_type=(
        jax.ShapeDtypeStruct((NC * AWORDS,), jnp.float32),
        jax.ShapeDtypeStruct((NE, NU), jnp.float32),
    ),
    mesh=plsc.VectorSubcoreMesh(core_axis_name="c", subcore_axis_name="s"),
    scratch_types=[
        pltpu.VMEM((HALF, ND), jnp.float32),
        pltpu.VMEM((HALF * NU,), jnp.float32),
        pltpu.VMEM((HALF * NU,), jnp.int32),
        pltpu.VMEM((HALF * NU,), jnp.float32),
        pltpu.VMEM((HALF * NU,), jnp.int32),
        pltpu.VMEM((HALF, NU), jnp.float32),
        pltpu.VMEM((CHUNK,), jnp.int32),
        pltpu.VMEM((CHUNK,), jnp.int32),
        pltpu.SemaphoreType.DMA,
        pltpu.SemaphoreType.DMA,
        pltpu.VMEM_SHARED((AWORDS,), jnp.float32),
    ],
)(_sc_scatter_body)


def _tc_sum_body(p_ref, o_ref):
    o_ref[...] = p_ref[0] + p_ref[1]


def _tc_sum(partials):
    cols = 128
    rows = AWORDS // cols
    blk = 784
    out = pl.pallas_call(
        _tc_sum_body,
        grid=(rows // blk,),
        in_specs=[pl.BlockSpec((NC, blk, cols), lambda i: (0, i, 0))],
        out_specs=pl.BlockSpec((blk, cols), lambda i: (i, 0)),
        out_shape=jax.ShapeDtypeStruct((rows, cols), jnp.float32),
    )(partials.reshape(NC, rows, cols))
    return out.reshape(AWORDS)[:NN * NU].reshape(NN, NU)


def kernel(unary, binary, deltas, index1, index2):
    idx1 = index1.astype(jnp.int32)
    idx2 = index2.astype(jnp.int32)
    partials, b = _sc_scatter(deltas, idx1, idx2)
    u = _tc_sum(partials)
    return (u, b)


# speedup vs baseline: 1.2616x; 1.2616x over previous
import functools

import jax
import jax.numpy as jnp
from jax import lax
from jax.experimental import pallas as pl
from jax.experimental.pallas import tpu as pltpu
from jax.experimental.pallas import tpu_sc as plsc

NU = 16
ND = 48
NN = 50000
NE = 1600000
NC, NS = 2, 16
NW = NC * NS
CHUNK = 256
HALF = 128
NCH = NE // CHUNK
CH_BASE = NCH // NW
CH_EXTRA = NCH - CH_BASE * NW
AWORDS = 802816
ZW = AWORDS // NS
_BCAST_DIMS = lax.GatherDimensionNumbers(
    offset_dims=(), collapsed_slice_dims=(0,), start_index_map=(0,))


def _sc_scatter_body(deltas, idx1, idx2, out, d_v, uf1_v, if1_v,
                     uf2_v, if2_v, i1_v, i2_v, sem_s, acc):
    c = lax.axis_index("c")
    s = lax.axis_index("s")
    g = c * NS + s

    zero16 = jnp.zeros((16,), jnp.float32)

    @plsc.parallel_loop(0, HALF * NU // 16, unroll=8)
    def _(i):
        uf1_v[pl.ds(i * 16, 16)] = zero16

    z0 = s * ZW
    for q in range(24):
        pltpu.sync_copy(uf1_v, acc.at[pl.ds(z0 + q * 2048, 2048)])
    pltpu.sync_copy(uf1_v.at[pl.ds(0, ZW - 24 * 2048)],
                    acc.at[pl.ds(z0 + 24 * 2048, ZW - 24 * 2048)])
    plsc.subcore_barrier()

    nch = CH_BASE + jnp.where(g < CH_EXTRA, 1, 0)
    start = g * CH_BASE + jnp.minimum(g, CH_EXTRA)
    iota16 = lax.iota(jnp.int32, 16)

    def chunk_body(i, carry):
        base = (start + i) * CHUNK
        pltpu.sync_copy(idx1.at[pl.ds(base, CHUNK)], i1_v)
        pltpu.sync_copy(idx2.at[pl.ds(base, CHUNK)], i2_v)
        for h in range(2):
            pltpu.sync_copy(deltas.at[pl.ds(base + h * HALF, HALF)], d_v)

            @plsc.parallel_loop(0, HALF, unroll=8)
            def _(r):
                grp = (r >> 4) << 4
                lane = r - grp
                bvec = jnp.full((16, 1), lane, jnp.int32)
                i1v = i1_v[pl.ds(h * HALF + grp, 16)]
                n1 = lax.gather(i1v, bvec, _BCAST_DIMS, (1,),
                                mode=lax.GatherScatterMode.PROMISE_IN_BOUNDS)
                i2v = i2_v[pl.ds(h * HALF + grp, 16)]
                n2 = lax.gather(i2v, bvec, _BCAST_DIMS, (1,),
                                mode=lax.GatherScatterMode.PROMISE_IN_BOUNDS)
                uf1_v[pl.ds(r * 16, 16)] = d_v[r, 0:NU]
                if1_v[pl.ds(r * 16, 16)] = n1 * NU + iota16
                uf2_v[pl.ds(r * 16, 16)] = d_v[r, NU:2 * NU]
                if2_v[pl.ds(r * 16, 16)] = n2 * NU + iota16

            d1 = pltpu.async_copy(uf1_v, acc.at[if1_v], sem_s, add=True)
            d2 = pltpu.async_copy(uf2_v, acc.at[if2_v], sem_s, add=True)
            d1.wait()
            d2.wait()
        return carry

    lax.fori_loop(0, nch, chunk_body, 0)
    plsc.subcore_barrier()

    pltpu.sync_copy(acc.at[pl.ds(z0, ZW)], out.at[pl.ds(c * AWORDS + z0, ZW)])


_sc_scatter = functools.partial(
    pl.kernel,
    out_type=jax.ShapeDtypeStruct((NC * AWORDS,), jnp.float32),
    mesh=plsc.VectorSubcoreMesh(core_axis_name="c", subcore_axis_name="s"),
    scratch_types=[
        pltpu.VMEM((HALF, ND), jnp.float32),
        pltpu.VMEM((HALF * NU,), jnp.float32),
        pltpu.VMEM((HALF * NU,), jnp.int32),
        pltpu.VMEM((HALF * NU,), jnp.float32),
        pltpu.VMEM((HALF * NU,), jnp.int32),
        pltpu.VMEM((CHUNK,), jnp.int32),
        pltpu.VMEM((CHUNK,), jnp.int32),
        pltpu.SemaphoreType.DMA,
        pltpu.VMEM_SHARED((AWORDS,), jnp.float32),
    ],
)(_sc_scatter_body)


def _tc_sum_body(p_ref, o_ref):
    o_ref[...] = p_ref[0] + p_ref[1]


def _tc_sum(partials):
    cols = 128
    rows = AWORDS // cols
    blk = 784
    out = pl.pallas_call(
        _tc_sum_body,
        grid=(rows // blk,),
        in_specs=[pl.BlockSpec((NC, blk, cols), lambda i: (0, i, 0))],
        out_specs=pl.BlockSpec((blk, cols), lambda i: (i, 0)),
        out_shape=jax.ShapeDtypeStruct((rows, cols), jnp.float32),
    )(partials.reshape(NC, rows, cols))
    return out.reshape(AWORDS)[:NN * NU].reshape(NN, NU)


def kernel(unary, binary, deltas, index1, index2):
    idx1 = index1.astype(jnp.int32)
    idx2 = index2.astype(jnp.int32)
    partials = _sc_scatter(deltas, idx1, idx2)
    u = _tc_sum(partials)
    b = deltas[:, 2 * NU:]
    return (u, b)
